# baseline (speedup 1.0000x reference)
import functools

import jax
import jax.numpy as jnp
from jax import lax
from jax.experimental import pallas as pl
from jax.experimental.pallas import tpu as pltpu
from jax.experimental.pallas import tpu_sc as plsc

CHUNK = 80
NBUF = 8


@functools.cache
def _build(n_rows, d_hid):
    info = plsc.get_sparse_core_info()
    nc, ns = info.num_cores, info.num_subcores
    nw = nc * ns
    n_chunks = n_rows // CHUNK
    assert n_chunks * CHUNK == n_rows
    chunks_per_w = n_chunks // nw
    assert chunks_per_w * nw == n_chunks
    n_iters = chunks_per_w // NBUF
    assert n_iters * NBUF == chunks_per_w

    mesh = plsc.VectorSubcoreMesh(core_axis_name="c", subcore_axis_name="s")

    @functools.partial(
        pl.kernel,
        out_type=jax.ShapeDtypeStruct((n_rows, d_hid), jnp.float32),
        mesh=mesh,
        scratch_types=[
            pltpu.VMEM((chunks_per_w, CHUNK), jnp.int32),
            pltpu.VMEM_SHARED((365, d_hid), jnp.float32),
            *[pltpu.VMEM((CHUNK, d_hid), jnp.float32) for _ in range(NBUF)],
            *[pltpu.SemaphoreType.DMA for _ in range(2 * NBUF)],
        ],
    )
    def gather(idx_hbm, table_hbm, out_hbm, idx_v, table_sh, *rest):
        rows = rest[:NBUF]
        gsem = rest[NBUF:2 * NBUF]
        ssem = rest[2 * NBUF:3 * NBUF]
        sid = lax.axis_index("s")
        wid = sid * nc + lax.axis_index("c")
        c0 = wid * chunks_per_w

        @pl.when(sid == 0)
        def _():
            pltpu.sync_copy(table_hbm, table_sh)

        pltpu.sync_copy(idx_hbm.at[pl.ds(c0, chunks_per_w)], idx_v)
        plsc.subcore_barrier()

        def g_copy(j, b):
            return pltpu.make_async_copy(
                table_sh.at[idx_v.at[j]], rows[b], gsem[b])

        def g_start(j, b):
            pltpu.async_copy(
                table_sh.at[idx_v.at[j]], rows[b], gsem[b], priority=1)

        def s_copy(j, b):
            return pltpu.make_async_copy(
                rows[b], out_hbm.at[pl.ds((c0 + j) * CHUNK, CHUNK)], ssem[b])

        for b in range(NBUF):
            g_start(b, b)

        @pl.loop(0, n_iters - 1)
        def _(i):
            for b in range(NBUF):
                j = i * NBUF + b
                g_copy(j, b).wait()
                s_copy(j, b).start()
            for b in range(NBUF):
                j = i * NBUF + b
                s_copy(j, b).wait()
                g_start(j + NBUF, b)

        last = n_iters - 1
        for b in range(NBUF):
            j = last * NBUF + b
            g_copy(j, b).wait()
            s_copy(j, b).start()
        for b in range(NBUF):
            s_copy(last * NBUF + b, b).wait()

    return gather


def kernel(doys, pos_table):
    b, l = doys.shape
    _, d = pos_table.shape
    n_rows = b * l
    idx2d = doys.astype(jnp.int32).reshape(n_rows // CHUNK, CHUNK)
    out = _build(n_rows, d)(idx2d, pos_table)
    return out.reshape(b, l, d)


# speedup vs baseline: 1.0005x; 1.0005x over previous
import functools

import jax
import jax.numpy as jnp
from jax import lax
from jax.experimental import pallas as pl
from jax.experimental.pallas import tpu as pltpu
from jax.experimental.pallas import tpu_sc as plsc

CHUNK = 80
NBUF = 8


@functools.cache
def _build(n_rows, d_hid):
    info = plsc.get_sparse_core_info()
    nc, ns = info.num_cores, info.num_subcores
    nw = nc * ns
    n_chunks = n_rows // CHUNK
    assert n_chunks * CHUNK == n_rows
    chunks_per_w = n_chunks // nw
    assert chunks_per_w * nw == n_chunks
    ngrp = NBUF // 2
    groups_per_w = chunks_per_w // 2
    n_iters = groups_per_w // ngrp
    assert n_iters * ngrp == groups_per_w

    mesh = plsc.VectorSubcoreMesh(core_axis_name="c", subcore_axis_name="s")

    @functools.partial(
        pl.kernel,
        out_type=jax.ShapeDtypeStruct((n_rows, d_hid), jnp.float32),
        mesh=mesh,
        scratch_types=[
            pltpu.VMEM((chunks_per_w, CHUNK), jnp.int32),
            pltpu.VMEM_SHARED((365, d_hid), jnp.float32),
            pltpu.VMEM((NBUF * CHUNK, d_hid), jnp.float32),
            *[pltpu.SemaphoreType.DMA for _ in range(NBUF + ngrp)],
        ],
    )
    def gather(idx_hbm, table_hbm, out_hbm, idx_v, table_sh, rows, *sems):
        gsem = sems[:NBUF]
        ssem = sems[NBUF:NBUF + ngrp]
        sid = lax.axis_index("s")
        wid = sid * nc + lax.axis_index("c")
        c0 = wid * chunks_per_w

        @pl.when(sid == 0)
        def _():
            pltpu.sync_copy(table_hbm, table_sh)

        pltpu.sync_copy(idx_hbm.at[pl.ds(c0, chunks_per_w)], idx_v)
        plsc.subcore_barrier()

        def g_copy(j, b):
            return pltpu.make_async_copy(
                table_sh.at[idx_v.at[j]],
                rows.at[pl.ds(b * CHUNK, CHUNK)], gsem[b])

        def s_copy(q, r):
            return pltpu.make_async_copy(
                rows.at[pl.ds(r * 2 * CHUNK, 2 * CHUNK)],
                out_hbm.at[pl.ds((c0 + 2 * q) * CHUNK, 2 * CHUNK)], ssem[r])

        for b in range(NBUF):
            g_copy(b, b).start()

        @pl.loop(0, n_iters - 1)
        def _(i):
            for r in range(ngrp):
                q = i * ngrp + r
                g_copy(2 * q, 2 * r).wait()
                g_copy(2 * q + 1, 2 * r + 1).wait()
                s_copy(q, r).start()
            for r in range(ngrp):
                q = i * ngrp + r
                s_copy(q, r).wait()
                g_copy(2 * q + NBUF, 2 * r).start()
                g_copy(2 * q + 1 + NBUF, 2 * r + 1).start()

        last = n_iters - 1
        for r in range(ngrp):
            q = last * ngrp + r
            g_copy(2 * q, 2 * r).wait()
            g_copy(2 * q + 1, 2 * r + 1).wait()
            s_copy(q, r).start()
        for r in range(ngrp):
            s_copy(last * ngrp + r, r).wait()

    return gather


def kernel(doys, pos_table):
    b, l = doys.shape
    _, d = pos_table.shape
    n_rows = b * l
    idx2d = doys.astype(jnp.int32).reshape(n_rows // CHUNK, CHUNK)
    out = _build(n_rows, d)(idx2d, pos_table)
    return out.reshape(b, l, d)
